# Initial kernel scaffold; baseline (speedup 1.0000x reference)
import jax
import jax.numpy as jnp
from jax.experimental import pallas as pl


def kernel(H):
    raise NotImplementedError("write your pallas kernel here")


# speedup vs baseline: 1.5989x; 1.5989x over previous
import jax
import jax.numpy as jnp
from jax.experimental import pallas as pl
from jax.experimental.pallas import tpu as pltpu

B = 4
W = 8
N = 16
TW = 4096 * 1024 // (W * N)
NSQ = N * N
K = NSQ // 2
NNZ = NSQ - K - N
EPS = 1e-8


def _copy_corr_kernel(x_ref, xn_ref, csum_ref):
    b = pl.program_id(0)
    w = pl.program_id(1)
    x = x_ref[0, 0]
    xn_ref[0] = x

    @pl.when(b == 0)
    def _():
        mean = jnp.mean(x, axis=1, keepdims=True)
        xc = x - mean
        cov = jax.lax.dot_general(
            xc, xc, (((1,), (1,)), ((), ())),
            preferred_element_type=jnp.float32,
        ) / (TW - 1 + EPS)
        rows = jax.lax.broadcasted_iota(jnp.int32, (N, N), 0)
        cols = jax.lax.broadcasted_iota(jnp.int32, (N, N), 1)
        eye = rows == cols
        var = jnp.sum(jnp.where(eye, cov, 0.0), axis=1, keepdims=True)
        std = jnp.sqrt(var + EPS)
        corr = jnp.clip(cov / (std * std.T + EPS), -1.0, 1.0)

        @pl.when(w == 0)
        def _():
            csum_ref[...] = corr

        @pl.when(w > 0)
        def _():
            csum_ref[...] += corr


def _edge_kernel(c_row_ref, c_col_ref, rows_ref, cols_ref):
    c_row = c_row_ref[...]
    c_col = c_col_ref[...]
    less = (c_col < c_row).astype(jnp.float32)
    rank = jnp.sum(less, axis=0, keepdims=True)

    fj = jax.lax.broadcasted_iota(jnp.int32, (1, NSQ), 1)
    r_j = fj // N
    c_j = fj % N
    keep = jnp.logical_and(rank >= K, r_j != c_j).astype(jnp.float32)

    ii = jax.lax.broadcasted_iota(jnp.int32, (NSQ, NSQ), 0)
    jj = jax.lax.broadcasted_iota(jnp.int32, (NSQ, NSQ), 1)
    upper = (ii < jj).astype(jnp.float32)
    pos = jax.lax.dot_general(
        keep, upper, (((1,), (0,)), ((), ())),
        preferred_element_type=jnp.float32,
    )

    slot = jax.lax.broadcasted_iota(jnp.int32, (NSQ, 1), 0).astype(jnp.float32)
    sel = (pos == slot).astype(jnp.float32) * keep
    fi = jax.lax.broadcasted_iota(jnp.int32, (NSQ, 1), 0)
    r_col = (fi // N).astype(jnp.float32)
    c_col_idx = (fi % N).astype(jnp.float32)
    rows_out = jax.lax.dot_general(
        sel, r_col, (((1,), (0,)), ((), ())),
        preferred_element_type=jnp.float32,
    )
    cols_out = jax.lax.dot_general(
        sel, c_col_idx, (((1,), (0,)), ((), ())),
        preferred_element_type=jnp.float32,
    )
    rows_ref[...] = rows_out.astype(jnp.int32)
    cols_ref[...] = cols_out.astype(jnp.int32)


def kernel(H):
    X = H.reshape(B, W, N, TW)
    x_nodes, csum = pl.pallas_call(
        _copy_corr_kernel,
        grid=(B, W),
        in_specs=[
            pl.BlockSpec((1, 1, N, TW), lambda b, w: (b, w, 0, 0)),
        ],
        out_specs=[
            pl.BlockSpec((1, N, TW), lambda b, w: (b, 0, w)),
            pl.BlockSpec((N, N), lambda b, w: (0, 0)),
        ],
        out_shape=[
            jax.ShapeDtypeStruct((B, N, W * TW), jnp.float32),
            jax.ShapeDtypeStruct((N, N), jnp.float32),
        ],
        compiler_params=pltpu.CompilerParams(
            dimension_semantics=("arbitrary", "arbitrary"),
        ),
    )(X)

    c_row = csum.reshape(1, NSQ)
    c_col = csum.reshape(NSQ, 1)
    rows, cols = pl.pallas_call(
        _edge_kernel,
        in_specs=[
            pl.BlockSpec((1, NSQ), lambda: (0, 0)),
            pl.BlockSpec((NSQ, 1), lambda: (0, 0)),
        ],
        out_specs=[
            pl.BlockSpec((NSQ, 1), lambda: (0, 0)),
            pl.BlockSpec((NSQ, 1), lambda: (0, 0)),
        ],
        out_shape=[
            jax.ShapeDtypeStruct((NSQ, 1), jnp.int32),
            jax.ShapeDtypeStruct((NSQ, 1), jnp.int32),
        ],
    )(c_row, c_col)

    edge_index = jnp.stack([rows[:NNZ, 0], cols[:NNZ, 0]], axis=0)
    return (x_nodes, edge_index)
